# baseline (speedup 1.0000x reference)
import functools

import numpy as np
import jax
import jax.numpy as jnp
from jax import lax
from jax.experimental import pallas as pl
from jax.experimental.pallas import tpu as pltpu
from jax.experimental.pallas import tpu_sc as plsc

_B = 4
_N = 63
_SENT = 32
_EMB = 16
_MEM = 64
_LOUT = 128
_VOCAB = 100000

_NI = _N + 1
_NSEG = _B * _NI * _MEM
_NQSEG = _B * _NI

_NW = 32
_SEG_PER_W = _NSEG // _NW
_CHUNK_SEGS = 4
_CHUNK_IDX = _CHUNK_SEGS * _SENT
_CHUNKS = _SEG_PER_W // _CHUNK_SEGS
_QSEG_PER_W = _NQSEG // _NW
_QCHUNKS = _QSEG_PER_W // _CHUNK_SEGS

_TV = 4096
_NVB = 25


def _pos_encoding(sentence_size, embedding_size):
    encoding = np.ones((embedding_size, sentence_size), dtype=np.float32)
    ls = sentence_size + 1
    le = embedding_size + 1
    for i in range(1, le):
        for j in range(1, ls):
            encoding[i - 1, j - 1] = (i - (le - 1) / 2) * (j - (ls - 1) / 2)
    encoding = 1 + 4 * encoding / embedding_size / sentence_size
    return np.transpose(encoding)


_ENC_NP = _pos_encoding(_SENT, _EMB)


_RING = 8


def _sc_body(tab_hbm, qtab_hbm, edge_hbm, node_hbm, graph_hbm, enc_hbm,
             outs_hbm, outq_hbm,
             idx_v, qidx_v, enc_v, ebuf, qbuf, *rest):
    bufs = list(rest[:_RING])
    qrows_a, qrows_b, out_v, qout_v = rest[_RING:_RING + 4]
    sems = list(rest[_RING + 4:2 * _RING + 4])
    qga, qgb = rest[2 * _RING + 4:]
    wid = lax.axis_index("c") * 16 + lax.axis_index("s")

    pltpu.sync_copy(enc_hbm, enc_v)

    zero16 = jnp.zeros((16,), jnp.int32)

    for qc in range(_QCHUNKS):
        for k in range(_CHUNK_SEGS):
            l = qc * _CHUNK_SEGS + k
            r = wid * _QSEG_PER_W + l
            b = r // _NI
            i = lax.rem(r, _NI)

            @pl.when(i < _N)
            def _():
                pltpu.sync_copy(node_hbm.at[b, i], qbuf)

            @pl.when(i == _N)
            def _():
                pltpu.sync_copy(graph_hbm.at[b], qbuf)

            qidx_v[qc, k * 32:k * 32 + 16] = qbuf[0:16].astype(jnp.int32)
            qidx_v[qc, k * 32 + 16:k * 32 + 32] = qbuf[16:32].astype(jnp.int32)

    def build_row(l, carry):
        r = wid * _QSEG_PER_W + l
        b = r // _NI
        i = lax.rem(r, _NI)

        @pl.when(i < _N)
        def _():
            pltpu.sync_copy(edge_hbm.at[b, i], ebuf)
            for cr in range(16):
                for g in range(8):
                    gg = cr * 8 + g
                    if gg < 126:
                        m, half = gg // 2, gg % 2
                        idx_v[16 * l + cr, 16 * g:16 * g + 16] = (
                            ebuf[m, 16 * half:16 * half + 16].astype(jnp.int32))
                    else:
                        idx_v[16 * l + cr, 16 * g:16 * g + 16] = zero16

        @pl.when(i == _N)
        def _():
            for cr in range(16):
                for g in range(8):
                    idx_v[16 * l + cr, 16 * g:16 * g + 16] = zero16

        return carry

    lax.fori_loop(0, _QSEG_PER_W, build_row, 0)

    pltpu.async_copy(qtab_hbm.at[qidx_v.at[0]], qrows_a, qga)
    pltpu.async_copy(qtab_hbm.at[qidx_v.at[1]], qrows_b, qgb)

    def fire(c, rows_ref, sem):
        pltpu.async_copy(tab_hbm.at[idx_v.at[c]], rows_ref, sem)

    def wait(rows_ref, sem):
        pltpu.make_async_copy(tab_hbm.at[idx_v.at[0]], rows_ref, sem).wait()

    def compute(c, rows_ref):
        am = [None] * _CHUNK_SEGS
        ao = [None] * _CHUNK_SEGS
        for s in range(_SENT):
            e = enc_v[s]
            for k in range(_CHUNK_SEGS):
                row = rows_ref[k * _SENT + s, :]
                sr, orow = plsc.unpack(row, format=plsc.PackFormat.INTERLEAVED)
                rm = sr * e
                ro = orow * e
                am[k] = rm if s == 0 else am[k] + rm
                ao[k] = ro if s == 0 else ao[k] + ro
        for k in range(_CHUNK_SEGS):
            seg = c * _CHUNK_SEGS + k
            out_v[seg, 0:16] = am[k]
            out_v[seg, 16:32] = ao[k]

    for r in range(_RING):
        fire(r, bufs[r], sems[r])

    def step(t, carry):
        for r in range(_RING):
            c = _RING * t + r
            wait(bufs[r], sems[r])
            compute(c, bufs[r])

            @pl.when(t < _CHUNKS // _RING - 1)
            def _():
                fire(c + _RING, bufs[r], sems[r])

        return carry

    lax.fori_loop(0, _CHUNKS // _RING, step, 0)

    def qcompute(qrows_ref, kbase):
        for k in range(_CHUNK_SEGS):
            acc0 = qrows_ref[k * _SENT + 0, :] * enc_v[0]
            acc1 = qrows_ref[k * _SENT + 1, :] * enc_v[1]
            for s in range(2, _SENT, 2):
                acc0 = acc0 + qrows_ref[k * _SENT + s, :] * enc_v[s]
                acc1 = acc1 + qrows_ref[k * _SENT + s + 1, :] * enc_v[s + 1]
            qout_v[kbase + k, :] = acc0 + acc1

    pltpu.make_async_copy(qtab_hbm.at[qidx_v.at[0]], qrows_a, qga).wait()
    qcompute(qrows_a, 0)
    pltpu.make_async_copy(qtab_hbm.at[qidx_v.at[1]], qrows_b, qgb).wait()
    qcompute(qrows_b, _CHUNK_SEGS)

    pltpu.sync_copy(out_v, outs_hbm.at[pl.ds(wid * _SEG_PER_W, _SEG_PER_W), :])
    pltpu.sync_copy(qout_v, outq_hbm.at[pl.ds(wid * _QSEG_PER_W, _QSEG_PER_W), :])


@functools.lru_cache(maxsize=None)
def _make_sc_call():
  return functools.partial(
    pl.kernel,
    out_type=[
        jax.ShapeDtypeStruct((_NSEG, 32), jnp.float32),
        jax.ShapeDtypeStruct((_NQSEG, _EMB), jnp.float32),
    ],
    mesh=plsc.VectorSubcoreMesh(core_axis_name="c", subcore_axis_name="s"),
    compiler_params=pltpu.CompilerParams(use_tc_tiling_on_sc=False,
                                         needs_layout_passes=False),
    scratch_types=[
        pltpu.VMEM((_CHUNKS, _CHUNK_IDX), jnp.int32),
        pltpu.VMEM((_QCHUNKS, _CHUNK_IDX), jnp.int32),
        pltpu.VMEM((_SENT, _EMB), jnp.float32),
        pltpu.VMEM((_N, _SENT), jnp.float32),
        pltpu.VMEM((_SENT,), jnp.float32),
        *([pltpu.VMEM((_CHUNK_IDX, 32), jnp.bfloat16)] * _RING),
        pltpu.VMEM((_CHUNK_IDX, _EMB), jnp.float32),
        pltpu.VMEM((_CHUNK_IDX, _EMB), jnp.float32),
        pltpu.VMEM((_SEG_PER_W, 32), jnp.float32),
        pltpu.VMEM((_QSEG_PER_W, _EMB), jnp.float32),
        *([pltpu.SemaphoreType.DMA] * (_RING + 2)),
    ],
  )(_sc_body)


def _tc_body(outs_ref, outq_ref, mb_ref, wout_ref, wfin_ref, o_ref, hf_scr):
    j = pl.program_id(0)
    b = pl.program_id(1)

    @pl.when((j == 0) & (b == 0))
    def _():
        mem = outs_ref[:, :, 0:16] + mb_ref[...][None, :, :]
        q = outq_ref[...]
        logits = jnp.sum(mem * q[:, None, :], axis=2)
        m = jnp.max(logits, axis=1, keepdims=True)
        e = jnp.exp(logits - m)
        probs = e / jnp.sum(e, axis=1, keepdims=True)
        outr = outs_ref[:, :, 16:32]
        lay = jnp.sum(outr * probs[:, :, None], axis=1)
        a = q + lay
        act = jnp.maximum(
            jnp.dot(a, wout_ref[...], preferred_element_type=jnp.float32), 0.0)
        r = act.reshape(_B, _NI, _LOUT)
        hf = r[:, :_N, :] + r[:, _NI - 1:_NI, :]
        hf_scr[...] = hf.reshape(_B * _N, _LOUT)

    o_ref[0] = jnp.dot(hf_scr[pl.ds(b * _N, _N), :], wfin_ref[...],
                       preferred_element_type=jnp.float32)


_tc_call = pl.pallas_call(
    _tc_body,
    grid=(_NVB, _B),
    in_specs=[
        pl.BlockSpec((_NQSEG, _MEM, 32), lambda j, b: (0, 0, 0)),
        pl.BlockSpec((_NQSEG, _EMB), lambda j, b: (0, 0)),
        pl.BlockSpec((_MEM, _EMB), lambda j, b: (0, 0)),
        pl.BlockSpec((_EMB, _LOUT), lambda j, b: (0, 0)),
        pl.BlockSpec((_LOUT, _TV), lambda j, b: (0, j)),
    ],
    out_specs=pl.BlockSpec((1, _N, _TV), lambda j, b: (b, 0, j)),
    out_shape=jax.ShapeDtypeStruct((_B, _N, _VOCAB), jnp.float32),
    scratch_shapes=[pltpu.VMEM((_B * _N, _LOUT), jnp.float32)],
)


def kernel(node_fts, edge_fts, graph_fts, adj_mat, hidden,
           query_biases, stories_biases, memory_contents, output_biases,
           w_output_linear, w_final):
    del hidden
    f32 = jnp.float32
    nil = jnp.zeros((1, _EMB), f32)
    tab = jnp.stack(
        [jnp.concatenate([stories_biases, nil], axis=0),
         jnp.concatenate([output_biases, nil], axis=0)],
        axis=2).reshape(_VOCAB, 2 * _EMB).astype(jnp.bfloat16)
    qtab = jnp.concatenate([query_biases, nil], axis=0)

    enc = jnp.asarray(_ENC_NP)

    outs, outq = _make_sc_call()(tab, qtab, edge_fts, node_fts, graph_fts, enc)
    outs = outs.reshape(_NQSEG, _MEM, 32)

    return _tc_call(outs, outq, memory_contents, w_output_linear, w_final)
